# baseline (speedup 1.0000x reference)
import functools

import jax
import jax.numpy as jnp
from jax import lax
from jax.experimental import pallas as pl
from jax.experimental.pallas import tpu as pltpu
from jax.experimental.pallas import tpu_sc as plsc

H = 128
_LN_EPS = 1e-5

_SC_INFO = plsc.get_sparse_core_info()
_NC = _SC_INFO.num_cores
_NS = _SC_INFO.num_subcores
_NW = _NC * _NS


def _mlp_body(nx2, has_xp, res_is_x1, *refs):
    i = 0
    x1_ref = refs[i]; i += 1
    x2_refs = refs[i:i + nx2]; i += nx2
    xp_ref = refs[i] if has_xp else None
    i += 1 if has_xp else 0
    a1_ref, a2_ref, b1_ref, w2_ref, b2_ref, g_ref, be_ref, o_ref = refs[i:]

    a = jnp.dot(x1_ref[...], a1_ref[...], preferred_element_type=jnp.float32)
    if nx2 == 1:
        a += jnp.dot(x2_refs[0][...], a2_ref[...],
                     preferred_element_type=jnp.float32)
    elif nx2 == 2:
        a += jnp.dot(x2_refs[0][...] + x2_refs[1][...], a2_ref[...],
                     preferred_element_type=jnp.float32)
    if has_xp:
        a += xp_ref[...]
    a += b1_ref[...]
    h = a * jax.nn.sigmoid(a)
    y = jnp.dot(h, w2_ref[...], preferred_element_type=jnp.float32)
    y += b2_ref[...]
    mu = jnp.mean(y, axis=-1, keepdims=True)
    yc = y - mu
    var = jnp.mean(yc * yc, axis=-1, keepdims=True)
    o = g_ref[...] * yc * lax.rsqrt(var + _LN_EPS) + be_ref[...]
    if res_is_x1:
        o += x1_ref[...].astype(jnp.float32)
    o_ref[...] = o


def _fused_mlp(x1, A1, b1, W2, b2, g, be, x2=None, A2=None, xp=None,
               res_is_x1=False, block=2000):
    n, k1 = x1.shape
    x2s = ()
    if x2 is not None:
        x2s = x2 if isinstance(x2, tuple) else (x2,)
    k2 = x2s[0].shape[1] if x2s else 1
    if A2 is None:
        A2 = jnp.zeros((k2, H), jnp.float32)
    grid = (pl.cdiv(n, block),)

    row_spec = lambda k: pl.BlockSpec((block, k), lambda i: (i, 0))
    full = lambda shp: pl.BlockSpec(shp, lambda i: tuple(0 for _ in shp))

    in_specs = [row_spec(k1)]
    operands = [x1]
    for xx in x2s:
        in_specs.append(row_spec(k2)); operands.append(xx)
    if xp is not None:
        in_specs.append(row_spec(H)); operands.append(xp)
    for w in (A1, A2):
        in_specs.append(full(w.shape))
    operands += [A1, A2]
    for v in (b1, W2, b2, g, be):
        vv = v.reshape((1, -1)) if v.ndim == 1 else v
        in_specs.append(full(vv.shape))
        operands.append(vv)

    body = functools.partial(_mlp_body, len(x2s), xp is not None,
                             res_is_x1)
    return pl.pallas_call(
        body,
        grid=grid,
        in_specs=in_specs,
        out_specs=pl.BlockSpec((block, H), lambda i: (i, 0)),
        out_shape=jax.ShapeDtypeStruct((n, H), jnp.float32),
    )(*operands)


def _proj_body(x_ref, wa_ref, wb_ref, oa_ref, ob_ref):
    x = x_ref[...]
    oa_ref[...] = jnp.dot(x, wa_ref[...], preferred_element_type=jnp.float32)
    ob_ref[...] = jnp.dot(x, wb_ref[...], preferred_element_type=jnp.float32)


def _proj1_body(x_ref, w_ref, o_ref):
    o_ref[...] = jnp.dot(x_ref[...], w_ref[...],
                         preferred_element_type=jnp.float32)


def _proj1(x, W, block=2000):
    n = x.shape[0]
    return pl.pallas_call(
        _proj1_body,
        grid=(pl.cdiv(n, block),),
        in_specs=[pl.BlockSpec((block, H), lambda i: (i, 0)),
                  pl.BlockSpec((H, H), lambda i: (0, 0))],
        out_specs=pl.BlockSpec((block, H), lambda i: (i, 0)),
        out_shape=jax.ShapeDtypeStruct((n, H), jnp.float32),
    )(x, W)


def _proj2(x, Wa, Wb, block=2000):
    n = x.shape[0]
    return pl.pallas_call(
        _proj_body,
        grid=(pl.cdiv(n, block),),
        in_specs=[pl.BlockSpec((block, H), lambda i: (i, 0)),
                  pl.BlockSpec((H, H), lambda i: (0, 0)),
                  pl.BlockSpec((H, H), lambda i: (0, 0))],
        out_specs=[pl.BlockSpec((block, H), lambda i: (i, 0)),
                   pl.BlockSpec((block, H), lambda i: (i, 0))],
        out_shape=[jax.ShapeDtypeStruct((n, H), jnp.float32),
                   jax.ShapeDtypeStruct((n, H), jnp.float32)],
    )(x, Wa, Wb)


_GC = 400


def _sc_mesh():
    return plsc.VectorSubcoreMesh(core_axis_name="c", subcore_axis_name="s")


def _vadd_rows(dst, src, nrows):
    def row(r, _):
        for c in range(H // 16):
            sl = pl.ds(c * 16, 16)
            plsc.addupdate(dst.at[r, sl], src[r, sl])
        return 0
    lax.fori_loop(0, nrows, row, 0)


_GG = 200


def _gather2(tableA, idxA, tableB, idxB):
    E = idxA.shape[0]
    nchunks = E // _GG
    assert E % _GG == 0

    @functools.partial(
        pl.kernel,
        mesh=_sc_mesh(),
        out_type=jax.ShapeDtypeStruct((E, H), jnp.float32),
        scratch_types=[
            pltpu.VMEM((_GG,), jnp.int32),
            pltpu.VMEM((_GG,), jnp.int32),
            pltpu.VMEM((_GG, H), jnp.float32),
            pltpu.VMEM((_GG, H), jnp.float32),
            pltpu.VMEM((_GG,), jnp.int32),
            pltpu.VMEM((_GG,), jnp.int32),
            pltpu.VMEM((_GG, H), jnp.float32),
            pltpu.VMEM((_GG, H), jnp.float32),
            pltpu.SemaphoreType.DMA,
            pltpu.SemaphoreType.DMA,
        ],
    )
    def k(tA, tB, iA, iB, out,
          ia0, ib0, ba0, bb0, ia1, ib1, ba1, bb1, sem0, sem1):
        wid = lax.axis_index("s") * _NC + lax.axis_index("c")
        trips = (nchunks - wid + _NW - 1) // _NW
        sets = ((ia0, ib0, ba0, bb0, sem0), (ia1, ib1, ba1, bb1, sem1))

        def issue(t, st):
            ia_v, ib_v, bufa, bufb, sem = st
            base = (wid + t * _NW) * _GG
            pltpu.sync_copy(iA.at[pl.ds(base, _GG)], ia_v)
            pltpu.sync_copy(iB.at[pl.ds(base, _GG)], ib_v)
            h1 = pltpu.async_copy(tA.at[ia_v], bufa, sem)
            h2 = pltpu.async_copy(tB.at[ib_v], bufb, sem)
            return h1, h2

        def finish(t, st, hs):
            _, _, bufa, bufb, _ = st
            base = (wid + t * _NW) * _GG
            hs[0].wait()
            hs[1].wait()
            _vadd_rows(bufa, bufb, _GG)
            pltpu.sync_copy(bufa, out.at[pl.ds(base, _GG)])

        def pair_body(u, _):
            t0 = 2 * u
            h0 = issue(t0, sets[0])
            h1 = issue(t0 + 1, sets[1])
            finish(t0, sets[0], h0)
            finish(t0 + 1, sets[1], h1)
            return 0

        lax.fori_loop(0, trips // 2, pair_body, 0)

        @pl.when(trips % 2 == 1)
        def _():
            t = trips - 1
            finish(t, sets[0], issue(t, sets[0]))

    return k(tableA, tableB, idxA, idxB)


_RC = 256


def _segsum(e, di, num_nodes):
    E = e.shape[0]
    nchunks = E // _GC
    assert E % _GC == 0
    lcap0 = _GC + 2 * _RC + 32
    scratch_words = _NS * (_GC + 2 * lcap0 + 2 * _RC + _RC * H)
    nrange = 2
    while True:
        R = -(-num_nodes // nrange // 8) * 8
        stripe = -(-R // _NS // 8) * 8
        acc_rows = _NS * stripe
        if acc_rows * H + scratch_words <= 2000000:
            break
        nrange += 2
    last_off = (_NS - 1) * stripe
    last_n = R - last_off
    assert 0 < last_n <= stripe and (R + 1) <= acc_rows
    lcap = _GC + 2 * _RC + 32

    @functools.partial(
        pl.kernel,
        mesh=_sc_mesh(),
        out_type=jax.ShapeDtypeStruct((nrange * R, H), jnp.float32),
        scratch_types=[
            pltpu.VMEM((_GC,), jnp.int32),
            pltpu.VMEM((lcap,), jnp.int32),
            pltpu.VMEM((lcap,), jnp.int32),
            pltpu.VMEM((_RC,), jnp.int32),
            pltpu.VMEM((_RC,), jnp.int32),
            pltpu.VMEM((_RC, H), jnp.float32),
            pltpu.VMEM_SHARED((acc_rows, H), jnp.float32),
            pltpu.SemaphoreType.DMA,
        ],
    )
    def k(ef, ii, out, idx_v, ids_l, rows_l, ids_c, rows_c, rows_v, acc, sem):
        cid = lax.axis_index("c")
        sid = lax.axis_index("s")
        trips = (nchunks - sid + _NS - 1) // _NS
        zt = jnp.zeros((16,), jnp.float32)
        lanes = lax.iota(jnp.int32, 16)

        def zrow(q, _):
            for c in range(H // 16):
                rows_v[q, pl.ds(c * 16, 16)] = zt
            return 0

        def fire(off):
            for g in range(_RC // 16):
                sl = pl.ds(g * 16, 16)
                ids_c[sl] = ids_l[pl.ds(off + g * 16, 16)]
                rows_c[sl] = rows_l[pl.ds(off + g * 16, 16)]
            pltpu.async_copy(ef.at[ids_c], rows_v, sem).wait()
            pltpu.sync_copy(rows_v, acc.at[rows_c], add=True)

        for rr in range(nrange // _NC):
            r = cid * (nrange // _NC) + rr
            lo = r * R
            lax.fori_loop(0, _RC, zrow, 0)
            off = 0
            while off < stripe:
                n = min(_RC, stripe - off)
                pltpu.sync_copy(rows_v.at[pl.ds(0, n)],
                                acc.at[pl.ds(sid * stripe + off, n)])
                off += n
            plsc.subcore_barrier()

            def chunk_body(t, count):
                base = (sid + t * _NS) * _GC
                pltpu.sync_copy(ii.at[pl.ds(base, _GC)], idx_v)
                for kk in range(_GC // 16):
                    iv = idx_v[pl.ds(kk * 16, 16)]
                    v = iv - lo
                    ok = (v >= 0) & (v < R)
                    gid = base + kk * 16 + lanes
                    s = jnp.where(ok, 1, 0)
                    for sh in (1, 2, 4, 8):
                        moved = jnp.take(s, jnp.maximum(lanes - sh, 0),
                                         axis=0)
                        s = s + jnp.where(lanes >= sh, moved, 0)
                    cnt = s[15]
                    pos = jnp.zeros((16,), jnp.int32)
                    for step in (8, 4, 2, 1):
                        pv = jnp.take(s, pos + (step - 1), axis=0)
                        pos = pos + jnp.where(pv <= lanes, step, 0)
                    gidx = jnp.minimum(pos, 15)
                    ids_l[pl.ds(count, 16)] = jnp.take(gid, gidx, axis=0)
                    rows_l[pl.ds(count, 16)] = jnp.take(v, gidx, axis=0)
                    count = count + cnt

                nfire = count // _RC

                def fire_body(f, _):
                    fire(f * _RC)
                    return 0

                lax.fori_loop(0, nfire, fire_body, 0)

                fbase = nfire * _RC
                for g in range(_RC // 16):
                    sl = pl.ds(g * 16, 16)
                    ids_l[sl] = ids_l[pl.ds(fbase + g * 16, 16)]
                    rows_l[sl] = rows_l[pl.ds(fbase + g * 16, 16)]
                return count - fbase

            count = lax.fori_loop(0, trips, chunk_body, jnp.int32(0))

            def pad_body(c, _):
                q = count + c * 16
                ids_l[pl.ds(q, 16)] = jnp.zeros((16,), jnp.int32)
                rows_l[pl.ds(q, 16)] = jnp.full((16,), R, jnp.int32)
                return 0

            lax.fori_loop(0, (_RC - count + 15) // 16, pad_body, 0)
            fire(0)
            plsc.subcore_barrier()

            @pl.when(sid < _NS - 1)
            def _():
                pltpu.sync_copy(
                    acc.at[pl.ds(sid * stripe, stripe)],
                    out.at[pl.ds(lo + sid * stripe, stripe)])

            @pl.when(sid == _NS - 1)
            def _():
                pltpu.sync_copy(
                    acc.at[pl.ds(last_off, last_n)],
                    out.at[pl.ds(lo + last_off, last_n)])

            plsc.subcore_barrier()

    full = k(e, di)
    return full[:num_nodes]


def _split_w1_edge(p):
    W1 = p["W1"]
    return W1[:H], W1[H:2 * H], W1[2 * H:]


def _edge_block(p, src, dst, e, si, di):
    W1e, W1s, W1d = _split_w1_edge(p)
    if src is dst:
        ps, pd = _proj2(src, W1s, W1d)
    else:
        ps = _proj1(src, W1s)
        pd = _proj1(dst, W1d)
    gathered = _gather2(ps, si, pd, di)
    return _fused_mlp(e, W1e, p["b1"], p["W2"], p["b2"], p["g"], p["be"],
                      xp=gathered, res_is_x1=True)


def _node_block(p, n, e, di, num_nodes):
    W1 = p["W1"]
    W1n, W1a = W1[:H], W1[H:]
    agg = _segsum(e, di, num_nodes)
    return _fused_mlp(n, W1n, p["b1"], p["W2"], p["b2"], p["g"], p["be"],
                      x2=agg, A2=W1a, res_is_x1=True)


def kernel(grid_node_features, mesh_node_features, mesh2mesh_edge_features,
           grid2mesh_edge_features, mesh2grid_edge_features, params,
           grid2mesh_edge_indices_src, grid2mesh_edge_indices_dst,
           mesh2mesh_edge_indices_src, mesh2mesh_edge_indices_dst,
           mesh2grid_edge_indices_src, mesh2grid_edge_indices_dst):
    p = params
    NG = grid_node_features.shape[0]
    NM = mesh_node_features.shape[0]

    def emb(name, x):
        q = p[name]
        return _fused_mlp(x, q["W1"], q["b1"], q["W2"], q["b2"], q["g"],
                          q["be"])

    g = emb("grid_emb", grid_node_features)
    m = emb("mesh_emb", mesh_node_features)
    e_g2m = emb("g2m_emb", grid2mesh_edge_features)
    e_m2g = emb("m2g_emb", mesh2grid_edge_features)
    e_m2m = emb("m2m_emb", mesh2mesh_edge_features)

    ef = _edge_block(p["enc_edge"], g, m, e_g2m,
                     grid2mesh_edge_indices_src, grid2mesh_edge_indices_dst)
    m = _node_block(p["enc_node"], m, ef, grid2mesh_edge_indices_dst, NM)
    q = p["enc_grid"]
    g = _fused_mlp(g, q["W1"], q["b1"], q["W2"], q["b2"], q["g"], q["be"],
                   res_is_x1=True)

    for l in range(len(p["proc_edge"])):
        e_m2m = _edge_block(p["proc_edge"][l], m, m, e_m2m,
                            mesh2mesh_edge_indices_src,
                            mesh2mesh_edge_indices_dst)
        m = _node_block(p["proc_node"][l], m, e_m2m,
                        mesh2mesh_edge_indices_dst, NM)

    ef = _edge_block(p["dec_edge"], m, g, e_m2g,
                     mesh2grid_edge_indices_src, mesh2grid_edge_indices_dst)
    g = _node_block(p["dec_node"], g, ef, mesh2grid_edge_indices_dst, NG)
    return g


# speedup vs baseline: 1.2028x; 1.2028x over previous
import functools

import jax
import jax.numpy as jnp
from jax import lax
from jax.experimental import pallas as pl
from jax.experimental.pallas import tpu as pltpu
from jax.experimental.pallas import tpu_sc as plsc

H = 128
_LN_EPS = 1e-5

_SC_INFO = plsc.get_sparse_core_info()
_NC = _SC_INFO.num_cores
_NS = _SC_INFO.num_subcores
_NW = _NC * _NS


def _mlp_body(nx2, has_xp, res_is_x1, *refs):
    i = 0
    x1_ref = refs[i]; i += 1
    x2_refs = refs[i:i + nx2]; i += nx2
    xp_ref = refs[i] if has_xp else None
    i += 1 if has_xp else 0
    a1_ref, a2_ref, b1_ref, w2_ref, b2_ref, g_ref, be_ref, o_ref = refs[i:]

    a = jnp.dot(x1_ref[...], a1_ref[...], preferred_element_type=jnp.float32)
    if nx2 == 1:
        a += jnp.dot(x2_refs[0][...], a2_ref[...],
                     preferred_element_type=jnp.float32)
    elif nx2 == 2:
        a += jnp.dot(x2_refs[0][...] + x2_refs[1][...], a2_ref[...],
                     preferred_element_type=jnp.float32)
    if has_xp:
        a += xp_ref[...]
    a += b1_ref[...]
    h = a * jax.nn.sigmoid(a)
    y = jnp.dot(h, w2_ref[...], preferred_element_type=jnp.float32)
    y += b2_ref[...]
    mu = jnp.mean(y, axis=-1, keepdims=True)
    yc = y - mu
    var = jnp.mean(yc * yc, axis=-1, keepdims=True)
    o = g_ref[...] * yc * lax.rsqrt(var + _LN_EPS) + be_ref[...]
    if res_is_x1:
        o += x1_ref[...].astype(jnp.float32)
    o_ref[...] = o


def _fused_mlp(x1, A1, b1, W2, b2, g, be, x2=None, A2=None, xp=None,
               res_is_x1=False, block=2000):
    n, k1 = x1.shape
    x2s = ()
    if x2 is not None:
        x2s = x2 if isinstance(x2, tuple) else (x2,)
    k2 = x2s[0].shape[1] if x2s else 1
    if A2 is None:
        A2 = jnp.zeros((k2, H), jnp.float32)
    grid = (pl.cdiv(n, block),)

    row_spec = lambda k: pl.BlockSpec((block, k), lambda i: (i, 0))
    full = lambda shp: pl.BlockSpec(shp, lambda i: tuple(0 for _ in shp))

    in_specs = [row_spec(k1)]
    operands = [x1]
    for xx in x2s:
        in_specs.append(row_spec(k2)); operands.append(xx)
    if xp is not None:
        in_specs.append(row_spec(H)); operands.append(xp)
    for w in (A1, A2):
        in_specs.append(full(w.shape))
    operands += [A1, A2]
    for v in (b1, W2, b2, g, be):
        vv = v.reshape((1, -1)) if v.ndim == 1 else v
        in_specs.append(full(vv.shape))
        operands.append(vv)

    body = functools.partial(_mlp_body, len(x2s), xp is not None,
                             res_is_x1)
    return pl.pallas_call(
        body,
        grid=grid,
        in_specs=in_specs,
        out_specs=pl.BlockSpec((block, H), lambda i: (i, 0)),
        out_shape=jax.ShapeDtypeStruct((n, H), jnp.float32),
    )(*operands)


def _proj_body(x_ref, wa_ref, wb_ref, oa_ref, ob_ref):
    x = x_ref[...]
    oa_ref[...] = jnp.dot(x, wa_ref[...], preferred_element_type=jnp.float32)
    ob_ref[...] = jnp.dot(x, wb_ref[...], preferred_element_type=jnp.float32)


def _proj1_body(x_ref, w_ref, o_ref):
    o_ref[...] = jnp.dot(x_ref[...], w_ref[...],
                         preferred_element_type=jnp.float32)


def _proj1(x, W, block=2000):
    n = x.shape[0]
    return pl.pallas_call(
        _proj1_body,
        grid=(pl.cdiv(n, block),),
        in_specs=[pl.BlockSpec((block, H), lambda i: (i, 0)),
                  pl.BlockSpec((H, H), lambda i: (0, 0))],
        out_specs=pl.BlockSpec((block, H), lambda i: (i, 0)),
        out_shape=jax.ShapeDtypeStruct((n, H), jnp.float32),
    )(x, W)


def _proj2(x, Wa, Wb, block=2000):
    n = x.shape[0]
    return pl.pallas_call(
        _proj_body,
        grid=(pl.cdiv(n, block),),
        in_specs=[pl.BlockSpec((block, H), lambda i: (i, 0)),
                  pl.BlockSpec((H, H), lambda i: (0, 0)),
                  pl.BlockSpec((H, H), lambda i: (0, 0))],
        out_specs=[pl.BlockSpec((block, H), lambda i: (i, 0)),
                   pl.BlockSpec((block, H), lambda i: (i, 0))],
        out_shape=[jax.ShapeDtypeStruct((n, H), jnp.float32),
                   jax.ShapeDtypeStruct((n, H), jnp.float32)],
    )(x, Wa, Wb)


_GC = 400


def _sc_mesh():
    return plsc.VectorSubcoreMesh(core_axis_name="c", subcore_axis_name="s")


def _vadd_rows(dst, src, nrows):
    def row(r, _):
        for c in range(H // 16):
            sl = pl.ds(c * 16, 16)
            plsc.addupdate(dst.at[r, sl], src[r, sl])
        return 0
    lax.fori_loop(0, nrows, row, 0)


_GG = 200


def _gather2(tableA, idxA, tableB, idxB):
    E = idxA.shape[0]
    nchunks = E // _GG
    assert E % _GG == 0

    @functools.partial(
        pl.kernel,
        mesh=_sc_mesh(),
        out_type=jax.ShapeDtypeStruct((E, H), jnp.float32),
        scratch_types=[
            pltpu.VMEM((_GG,), jnp.int32),
            pltpu.VMEM((_GG,), jnp.int32),
            pltpu.VMEM((_GG, H), jnp.float32),
            pltpu.VMEM((_GG, H), jnp.float32),
            pltpu.VMEM((_GG,), jnp.int32),
            pltpu.VMEM((_GG,), jnp.int32),
            pltpu.VMEM((_GG, H), jnp.float32),
            pltpu.VMEM((_GG, H), jnp.float32),
            pltpu.SemaphoreType.DMA,
            pltpu.SemaphoreType.DMA,
        ],
    )
    def k(tA, tB, iA, iB, out,
          ia0, ib0, ba0, bb0, ia1, ib1, ba1, bb1, sem0, sem1):
        wid = lax.axis_index("s") * _NC + lax.axis_index("c")
        trips = (nchunks - wid + _NW - 1) // _NW
        sets = ((ia0, ib0, ba0, bb0, sem0), (ia1, ib1, ba1, bb1, sem1))

        def issue(t, st):
            ia_v, ib_v, bufa, bufb, sem = st
            base = (wid + t * _NW) * _GG
            pltpu.sync_copy(iA.at[pl.ds(base, _GG)], ia_v)
            pltpu.sync_copy(iB.at[pl.ds(base, _GG)], ib_v)
            h1 = pltpu.async_copy(tA.at[ia_v], bufa, sem)
            h2 = pltpu.async_copy(tB.at[ib_v], bufb, sem)
            return h1, h2

        def finish(t, st, hs):
            _, _, bufa, bufb, _ = st
            base = (wid + t * _NW) * _GG
            hs[0].wait()
            hs[1].wait()
            _vadd_rows(bufa, bufb, _GG)
            pltpu.sync_copy(bufa, out.at[pl.ds(base, _GG)])

        def pair_body(u, _):
            t0 = 2 * u
            h0 = issue(t0, sets[0])
            h1 = issue(t0 + 1, sets[1])
            finish(t0, sets[0], h0)
            finish(t0 + 1, sets[1], h1)
            return 0

        lax.fori_loop(0, trips // 2, pair_body, 0)

        @pl.when(trips % 2 == 1)
        def _():
            t = trips - 1
            finish(t, sets[0], issue(t, sets[0]))

    return k(tableA, tableB, idxA, idxB)


_RC = 128


def _segsum(e, di, num_nodes):
    E = e.shape[0]
    nchunks = E // _GC
    assert E % _GC == 0
    lcap0 = _GC + 2 * _RC + 32
    scratch_words = _NS * (_GC + 2 * lcap0 + 2 * _RC + _RC * H)
    nrange = 2
    while True:
        R = -(-num_nodes // nrange // 8) * 8
        stripe = -(-R // _NS // 8) * 8
        acc_rows = _NS * stripe
        if acc_rows * H + scratch_words <= 2000000:
            break
        nrange += 2
    last_off = (_NS - 1) * stripe
    last_n = R - last_off
    assert 0 < last_n <= stripe and (R + 1) <= acc_rows
    lcap = _GC + 2 * _RC + 32

    @functools.partial(
        pl.kernel,
        mesh=_sc_mesh(),
        out_type=jax.ShapeDtypeStruct((nrange * R, H), jnp.float32),
        scratch_types=[
            pltpu.VMEM((_GC,), jnp.int32),
            pltpu.VMEM((lcap,), jnp.int32),
            pltpu.VMEM((lcap,), jnp.int32),
            pltpu.VMEM((_RC,), jnp.int32),
            pltpu.VMEM((_RC,), jnp.int32),
            pltpu.VMEM((_RC, H), jnp.float32),
            pltpu.VMEM_SHARED((acc_rows, H), jnp.float32),
            pltpu.SemaphoreType.DMA,
        ],
    )
    def k(ef, ii, out, idx_v, ids_l, rows_l, ids_c, rows_c, rows_v, acc, sem):
        cid = lax.axis_index("c")
        sid = lax.axis_index("s")
        trips = (nchunks - sid + _NS - 1) // _NS
        zt = jnp.zeros((16,), jnp.float32)
        lanes = lax.iota(jnp.int32, 16)

        def zrow(q, _):
            for c in range(H // 16):
                rows_v[q, pl.ds(c * 16, 16)] = zt
            return 0

        def fire(off):
            for g in range(_RC // 16):
                sl = pl.ds(g * 16, 16)
                ids_c[sl] = ids_l[pl.ds(off + g * 16, 16)]
                rows_c[sl] = rows_l[pl.ds(off + g * 16, 16)]
            pltpu.async_copy(ef.at[ids_c], rows_v, sem).wait()
            pltpu.sync_copy(rows_v, acc.at[rows_c], add=True)

        for rr in range(nrange // _NC):
            r = cid * (nrange // _NC) + rr
            lo = r * R
            lax.fori_loop(0, _RC, zrow, 0)
            off = 0
            while off < stripe:
                n = min(_RC, stripe - off)
                pltpu.sync_copy(rows_v.at[pl.ds(0, n)],
                                acc.at[pl.ds(sid * stripe + off, n)])
                off += n
            plsc.subcore_barrier()

            def chunk_body(t, count):
                base = (sid + t * _NS) * _GC
                pltpu.sync_copy(ii.at[pl.ds(base, _GC)], idx_v)
                for kk in range(_GC // 16):
                    iv = idx_v[pl.ds(kk * 16, 16)]
                    v = iv - lo
                    ok = (v >= 0) & (v < R)
                    gid = base + kk * 16 + lanes
                    s = jnp.where(ok, 1, 0)
                    for sh in (1, 2, 4, 8):
                        moved = jnp.take(s, jnp.maximum(lanes - sh, 0),
                                         axis=0)
                        s = s + jnp.where(lanes >= sh, moved, 0)
                    cnt = s[15]
                    pos = jnp.zeros((16,), jnp.int32)
                    for step in (8, 4, 2, 1):
                        pv = jnp.take(s, pos + (step - 1), axis=0)
                        pos = pos + jnp.where(pv <= lanes, step, 0)
                    gidx = jnp.minimum(pos, 15)
                    ids_l[pl.ds(count, 16)] = jnp.take(gid, gidx, axis=0)
                    rows_l[pl.ds(count, 16)] = jnp.take(v, gidx, axis=0)
                    count = count + cnt

                nfire = count // _RC

                def fire_body(f, _):
                    fire(f * _RC)
                    return 0

                lax.fori_loop(0, nfire, fire_body, 0)

                fbase = nfire * _RC
                for g in range(_RC // 16):
                    sl = pl.ds(g * 16, 16)
                    ids_l[sl] = ids_l[pl.ds(fbase + g * 16, 16)]
                    rows_l[sl] = rows_l[pl.ds(fbase + g * 16, 16)]
                return count - fbase

            count = lax.fori_loop(0, trips, chunk_body, jnp.int32(0))

            def pad_body(c, _):
                q = count + c * 16
                ids_l[pl.ds(q, 16)] = jnp.zeros((16,), jnp.int32)
                rows_l[pl.ds(q, 16)] = jnp.full((16,), R, jnp.int32)
                return 0

            lax.fori_loop(0, (_RC - count + 15) // 16, pad_body, 0)
            fire(0)
            plsc.subcore_barrier()

            @pl.when(sid < _NS - 1)
            def _():
                pltpu.sync_copy(
                    acc.at[pl.ds(sid * stripe, stripe)],
                    out.at[pl.ds(lo + sid * stripe, stripe)])

            @pl.when(sid == _NS - 1)
            def _():
                pltpu.sync_copy(
                    acc.at[pl.ds(last_off, last_n)],
                    out.at[pl.ds(lo + last_off, last_n)])

            plsc.subcore_barrier()

    full = k(e, di)
    return full[:num_nodes]


def _split_w1_edge(p):
    W1 = p["W1"]
    return W1[:H], W1[H:2 * H], W1[2 * H:]


def _edge_block(p, src, dst, e, si, di):
    W1e, W1s, W1d = _split_w1_edge(p)
    if src is dst:
        ps, pd = _proj2(src, W1s, W1d)
    else:
        ps = _proj1(src, W1s)
        pd = _proj1(dst, W1d)
    gathered = _gather2(ps, si, pd, di)
    return _fused_mlp(e, W1e, p["b1"], p["W2"], p["b2"], p["g"], p["be"],
                      xp=gathered, res_is_x1=True)


def _node_block(p, n, e, di, num_nodes):
    W1 = p["W1"]
    W1n, W1a = W1[:H], W1[H:]
    agg = _segsum(e, di, num_nodes)
    return _fused_mlp(n, W1n, p["b1"], p["W2"], p["b2"], p["g"], p["be"],
                      x2=agg, A2=W1a, res_is_x1=True)


def kernel(grid_node_features, mesh_node_features, mesh2mesh_edge_features,
           grid2mesh_edge_features, mesh2grid_edge_features, params,
           grid2mesh_edge_indices_src, grid2mesh_edge_indices_dst,
           mesh2mesh_edge_indices_src, mesh2mesh_edge_indices_dst,
           mesh2grid_edge_indices_src, mesh2grid_edge_indices_dst):
    p = params
    NG = grid_node_features.shape[0]
    NM = mesh_node_features.shape[0]

    def emb(name, x):
        q = p[name]
        return _fused_mlp(x, q["W1"], q["b1"], q["W2"], q["b2"], q["g"],
                          q["be"])

    g = emb("grid_emb", grid_node_features)
    m = emb("mesh_emb", mesh_node_features)
    e_g2m = emb("g2m_emb", grid2mesh_edge_features)
    e_m2g = emb("m2g_emb", mesh2grid_edge_features)
    e_m2m = emb("m2m_emb", mesh2mesh_edge_features)

    ef = _edge_block(p["enc_edge"], g, m, e_g2m,
                     grid2mesh_edge_indices_src, grid2mesh_edge_indices_dst)
    m = _node_block(p["enc_node"], m, ef, grid2mesh_edge_indices_dst, NM)
    q = p["enc_grid"]
    g = _fused_mlp(g, q["W1"], q["b1"], q["W2"], q["b2"], q["g"], q["be"],
                   res_is_x1=True)

    for l in range(len(p["proc_edge"])):
        e_m2m = _edge_block(p["proc_edge"][l], m, m, e_m2m,
                            mesh2mesh_edge_indices_src,
                            mesh2mesh_edge_indices_dst)
        m = _node_block(p["proc_node"][l], m, e_m2m,
                        mesh2mesh_edge_indices_dst, NM)

    ef = _edge_block(p["dec_edge"], m, g, e_m2g,
                     mesh2grid_edge_indices_src, mesh2grid_edge_indices_dst)
    g = _node_block(p["dec_node"], g, ef, mesh2grid_edge_indices_dst, NG)
    return g
